# baseline (speedup 1.0000x reference)
import functools

import jax
import jax.numpy as jnp
from jax import lax
from jax.experimental import pallas as pl
from jax.experimental.pallas import tpu as pltpu
from jax.experimental.pallas import tpu_sc as plsc

_NC = 2
_NS = 16
_NW = _NC * _NS
_D = 64
_B = 16384
_RPW = _B // _NW
_CHUNK = 128
_NCH = _RPW // _CHUNK
_L = 16

_mesh = plsc.VectorSubcoreMesh(core_axis_name="c", subcore_axis_name="s")


@functools.partial(
    pl.kernel,
    mesh=_mesh,
    out_type=jax.ShapeDtypeStruct((_B, _D), jnp.float32),
    scratch_types=[
        pltpu.VMEM((_RPW,), jnp.int32),
        pltpu.VMEM((_RPW,), jnp.int32),
        pltpu.VMEM((2, _CHUNK, _D), jnp.float32),
        pltpu.SemaphoreType.DMA,
        pltpu.SemaphoreType.DMA,
    ],
)
def _multi_index_select(idxf_hbm, idxt_hbm, mat1_hbm, mat2_hbm, out_hbm,
                        idxf_s, idxt_s, rows_v, gsem, ssem):
    wid = lax.axis_index("s") * _NC + lax.axis_index("c")
    pltpu.sync_copy(idxf_hbm.at[wid], idxf_s)
    pltpu.sync_copy(idxt_hbm.at[wid], idxt_s)

    def _move(mat_hbm):
        def gather_chunk(j, buf):
            def issue16(g, _):
                v = idxf_s[pl.ds(j * _CHUNK + g * _L, _L)]
                for i in range(_L):
                    pltpu.async_copy(mat_hbm.at[pl.ds(v[i], 1)],
                                     rows_v.at[buf].at[pl.ds(g * _L + i, 1)],
                                     gsem)
                return _
            lax.fori_loop(0, _CHUNK // _L, issue16, 0)

        def drain_gather(buf):
            pltpu.make_async_copy(mat_hbm.at[pl.ds(0, _CHUNK)],
                                  rows_v.at[buf], gsem).wait()

        def scatter_chunk(j, buf):
            def issue16(g, _):
                v = idxt_s[pl.ds(j * _CHUNK + g * _L, _L)]
                for i in range(_L):
                    pltpu.async_copy(rows_v.at[buf].at[pl.ds(g * _L + i, 1)],
                                     out_hbm.at[pl.ds(v[i], 1)], ssem)
                return _
            lax.fori_loop(0, _CHUNK // _L, issue16, 0)

        def drain_scatter():
            pltpu.make_async_copy(mat_hbm.at[pl.ds(0, _CHUNK)],
                                  rows_v.at[0], ssem).wait()

        gather_chunk(0, 0)
        for j in range(_NCH):
            buf = j % 2
            drain_gather(buf)
            if j + 1 < _NCH:
                gather_chunk(j + 1, (j + 1) % 2)
            scatter_chunk(j, buf)
        for _ in range(_NCH):
            drain_scatter()

    @pl.when(wid < _NW // 2)
    def _():
        _move(mat1_hbm)

    @pl.when(wid >= _NW // 2)
    def _():
        _move(mat2_hbm)


def kernel(idx_froms, idx_tos, mat1, mat2):
    idxf = idx_froms.reshape(_NW, _RPW)
    idxt = idx_tos.reshape(_NW, _RPW)
    return _multi_index_select(idxf, idxt, mat1, mat2)


# speedup vs baseline: 1.0008x; 1.0008x over previous
import functools

import jax
import jax.numpy as jnp
from jax import lax
from jax.experimental import pallas as pl
from jax.experimental.pallas import tpu as pltpu
from jax.experimental.pallas import tpu_sc as plsc

_NC = 2
_NS = 16
_NW = _NC * _NS
_D = 64
_B = 16384
_RPW = _B // _NW
_CHUNK = 128
_NCH = _RPW // _CHUNK
_L = 16

_mesh = plsc.VectorSubcoreMesh(core_axis_name="c", subcore_axis_name="s")


@functools.partial(
    pl.kernel,
    mesh=_mesh,
    out_type=jax.ShapeDtypeStruct((_B, _D), jnp.float32),
    scratch_types=[
        pltpu.VMEM((_RPW,), jnp.int32),
        pltpu.VMEM((_RPW,), jnp.int32),
        pltpu.VMEM((2, _CHUNK, _D), jnp.float32),
        pltpu.SemaphoreType.DMA,
        pltpu.SemaphoreType.DMA,
    ],
)
def _multi_index_select(idxf_hbm, idxt_hbm, mat1_hbm, mat2_hbm, out_hbm,
                        idxf_s, idxt_s, rows_v, gsem, ssem):
    wid = lax.axis_index("s") * _NC + lax.axis_index("c")
    pltpu.sync_copy(idxf_hbm.at[wid], idxf_s)
    pltpu.sync_copy(idxt_hbm.at[wid], idxt_s)

    def _move(mat_hbm):
        def gather_chunk(j, buf):
            def issue16(g, _):
                v = idxf_s[pl.ds(j * _CHUNK + g * _L, _L)]
                for i in range(_L):
                    pltpu.async_copy(mat_hbm.at[pl.ds(v[i], 1)],
                                     rows_v.at[buf].at[pl.ds(g * _L + i, 1)],
                                     gsem)
                return _
            lax.fori_loop(0, _CHUNK // _L, issue16, 0)

        def drain_gather(buf):
            pltpu.make_async_copy(mat_hbm.at[pl.ds(0, _CHUNK)],
                                  rows_v.at[buf], gsem).wait()

        def scatter_chunk(j, buf):
            def issue16(g, _):
                v = idxt_s[pl.ds(j * _CHUNK + g * _L, _L)]
                for i in range(_L):
                    pltpu.async_copy(rows_v.at[buf].at[pl.ds(g * _L + i, 1)],
                                     out_hbm.at[pl.ds(v[i], 1)], ssem)
                return _
            lax.fori_loop(0, _CHUNK // _L, issue16, 0)

        def drain_scatter():
            pltpu.make_async_copy(mat_hbm.at[pl.ds(0, _CHUNK)],
                                  rows_v.at[0], ssem).wait()

        gather_chunk(0, 0)
        for j in range(_NCH):
            buf = j % 2
            drain_gather(buf)
            if j + 1 < _NCH:
                if j >= 1:
                    drain_scatter()
                gather_chunk(j + 1, (j + 1) % 2)
            scatter_chunk(j, buf)
        drain_scatter()
        drain_scatter()

    @pl.when(wid < _NW // 2)
    def _():
        _move(mat1_hbm)

    @pl.when(wid >= _NW // 2)
    def _():
        _move(mat2_hbm)


def kernel(idx_froms, idx_tos, mat1, mat2):
    idxf = idx_froms.reshape(_NW, _RPW)
    idxt = idx_tos.reshape(_NW, _RPW)
    return _multi_index_select(idxf, idxt, mat1, mat2)
